# baseline (device time: 20579 ns/iter reference)
import jax
import jax.numpy as jnp
from jax import lax
from jax.experimental import pallas as pl
from jax.experimental.pallas import tpu as pltpu

N_DEV = 4
B, Sq, Skv, HQ_GLOBAL, Dh = 2, 256, 256, 16, 64
H = HQ_GLOBAL // N_DEV
D_MODEL = 512
SCALE = 0.125

_A_S1, _A_R1, _A_S2, _A_R2, _B_S1, _B_R1, _B_S2, _B_R2 = range(8)


def _attn_partial(q, k_b, v_b, wo, mask, b):
    acc = jnp.zeros((Sq, D_MODEL), jnp.float32)
    for h in range(H):
        qbh = q[b * Sq:(b + 1) * Sq, h * Dh:(h + 1) * Dh]
        kbh = k_b[:, h * Dh:(h + 1) * Dh]
        vbh = v_b[:, h * Dh:(h + 1) * Dh]
        scores = lax.dot_general(
            qbh, kbh,
            dimension_numbers=(((1,), (1,)), ((), ())),
            preferred_element_type=jnp.float32)
        scores = jnp.where(mask, scores, -1e9)
        w = jnp.exp(scores - jnp.max(scores, axis=-1, keepdims=True))
        inv = 1.0 / jnp.sum(w, axis=-1, keepdims=True)
        ctx = jnp.dot(w.astype(jnp.bfloat16), vbh,
                      preferred_element_type=jnp.float32) * inv
        acc = acc + jnp.dot(
            ctx.astype(jnp.bfloat16), wo[h * Dh:(h + 1) * Dh, :],
            preferred_element_type=jnp.float32)
    return acc


def _exchange(comm_ref, send_sems, recv_sems, src_slot, dst_slot, sem, peer):
    rdma = pltpu.make_async_remote_copy(
        src_ref=comm_ref.at[src_slot],
        dst_ref=comm_ref.at[dst_slot],
        send_sem=send_sems.at[sem],
        recv_sem=recv_sems.at[sem],
        device_id=(peer,),
        device_id_type=pl.DeviceIdType.MESH,
    )
    rdma.start()
    return rdma


def _body(x_ref, wq_ref, k_ref, v_ref, wo_ref, out_ref,
          comm_ref, send_sems, recv_sems):
    my_pos = lax.axis_index("i")
    yp = my_pos ^ 1
    xp = 3 - my_pos

    barrier_sem = pltpu.get_barrier_semaphore()
    for peer in (yp, xp):
        pl.semaphore_signal(barrier_sem, inc=1, device_id=(peer,),
                            device_id_type=pl.DeviceIdType.MESH)
    pl.semaphore_wait(barrier_sem, 2)

    q = jnp.dot(x_ref[...].astype(jnp.bfloat16),
                wq_ref[...].astype(jnp.bfloat16),
                preferred_element_type=jnp.float32)
    q = (q * SCALE).astype(jnp.bfloat16)
    wo = wo_ref[...].astype(jnp.bfloat16)
    off = my_pos * H * Dh

    qi = lax.broadcasted_iota(jnp.int32, (Sq, Skv), 0)
    ki = lax.broadcasted_iota(jnp.int32, (Sq, Skv), 1)
    mask = (jnp.abs(qi - ki) <= 128) | (ki < 32) | (qi < 32)

    acc0 = _attn_partial(q, k_ref[0, :, pl.ds(off, H * Dh)].astype(jnp.bfloat16),
                         v_ref[0, :, pl.ds(off, H * Dh)].astype(jnp.bfloat16),
                         wo, mask, 0)
    comm_ref[_A_S1] = acc0.astype(jnp.bfloat16)
    a1 = _exchange(comm_ref, send_sems, recv_sems, _A_S1, _A_R1, 0, yp)

    acc1 = _attn_partial(q, k_ref[1, :, pl.ds(off, H * Dh)].astype(jnp.bfloat16),
                         v_ref[1, :, pl.ds(off, H * Dh)].astype(jnp.bfloat16),
                         wo, mask, 1)
    comm_ref[_B_S1] = acc1.astype(jnp.bfloat16)
    b1 = _exchange(comm_ref, send_sems, recv_sems, _B_S1, _B_R1, 1, xp)

    a1.wait()
    sum_a = acc0 + comm_ref[_A_R1].astype(jnp.float32)
    comm_ref[_A_S2] = sum_a.astype(jnp.bfloat16)
    a2 = _exchange(comm_ref, send_sems, recv_sems, _A_S2, _A_R2, 2, xp)

    b1.wait()
    sum_b = acc1 + comm_ref[_B_R1].astype(jnp.float32)
    comm_ref[_B_S2] = sum_b.astype(jnp.bfloat16)
    b2 = _exchange(comm_ref, send_sems, recv_sems, _B_S2, _B_R2, 3, yp)

    a2.wait()
    out_ref[0] = sum_a + comm_ref[_A_R2].astype(jnp.float32)
    b2.wait()
    out_ref[1] = sum_b + comm_ref[_B_R2].astype(jnp.float32)


def kernel(x, Wq, K_ext, V_ext, Wo):
    K2 = K_ext.reshape(B, Skv, HQ_GLOBAL * Dh)
    V2 = V_ext.reshape(B, Skv, HQ_GLOBAL * Dh)
    x2 = x.reshape(B * Sq, D_MODEL)

    return pl.pallas_call(
        _body,
        out_shape=jax.ShapeDtypeStruct((B, Sq, D_MODEL), jnp.float32),
        in_specs=[pl.BlockSpec(memory_space=pltpu.VMEM)] * 5,
        out_specs=pl.BlockSpec(memory_space=pltpu.VMEM),
        scratch_shapes=[
            pltpu.VMEM((8, Sq, D_MODEL), jnp.bfloat16),
            pltpu.SemaphoreType.DMA((4,)),
            pltpu.SemaphoreType.DMA((4,)),
        ],
        compiler_params=pltpu.CompilerParams(collective_id=0),
    )(x2, Wq, K2, V2, Wo)


# device time: 17255 ns/iter; 1.1926x vs baseline; 1.1926x over previous
import jax
import jax.numpy as jnp
from jax import lax
from jax.experimental import pallas as pl
from jax.experimental.pallas import tpu as pltpu

N_DEV = 4
B, Sq, Skv, HQ_GLOBAL, Dh = 2, 256, 256, 16, 64
H = HQ_GLOBAL // N_DEV
D_MODEL = 512
SCALE = 0.125

R = 128
CHUNKS = [(b, r) for b in range(B) for r in range(Sq // R)]


def _attn_chunk(q_c, k_b, v_b, wo, bias_c):
    acc = jnp.zeros((R, D_MODEL), jnp.float32)
    for h in range(H):
        qbh = q_c[:, h * Dh:(h + 1) * Dh]
        kbh = k_b[:, h * Dh:(h + 1) * Dh]
        vbh = v_b[:, h * Dh:(h + 1) * Dh]
        scores = lax.dot_general(
            qbh, kbh,
            dimension_numbers=(((1,), (1,)), ((), ())),
            preferred_element_type=jnp.float32)
        w = jnp.exp(scores + bias_c)
        inv = 1.0 / jnp.sum(w, axis=-1, keepdims=True)
        ctx = jnp.dot(w.astype(jnp.bfloat16), vbh,
                      preferred_element_type=jnp.float32) * inv
        acc = acc + jnp.dot(
            ctx.astype(jnp.bfloat16), wo[h * Dh:(h + 1) * Dh, :],
            preferred_element_type=jnp.float32)
    return acc


def _exchange(comm_ref, send_sems, recv_sems, src_slot, dst_slot, sem, peer):
    rdma = pltpu.make_async_remote_copy(
        src_ref=comm_ref.at[src_slot],
        dst_ref=comm_ref.at[dst_slot],
        send_sem=send_sems.at[sem],
        recv_sem=recv_sems.at[sem],
        device_id=(peer,),
        device_id_type=pl.DeviceIdType.MESH,
    )
    rdma.start()
    return rdma


def _body(x_ref, wq_ref, k_ref, v_ref, wo_ref, out_ref,
          comm_ref, send_sems, recv_sems):
    my_pos = lax.axis_index("i")
    yp = my_pos ^ 1
    xp = 3 - my_pos

    barrier_sem = pltpu.get_barrier_semaphore()
    for peer in (yp, xp):
        pl.semaphore_signal(barrier_sem, inc=1, device_id=(peer,),
                            device_id_type=pl.DeviceIdType.MESH)

    wq = wq_ref[...].astype(jnp.bfloat16)
    wo = wo_ref[...].astype(jnp.bfloat16)
    off = my_pos * H * Dh

    qi = lax.broadcasted_iota(jnp.int32, (Sq, Skv), 0)
    ki = lax.broadcasted_iota(jnp.int32, (Sq, Skv), 1)
    mask = (jnp.abs(qi - ki) <= 128) | (ki < 32) | (qi < 32)
    bias = jnp.where(mask, 0.0, -1e9).astype(jnp.float32)

    kv = {}
    for b in range(B):
        kv[b] = (k_ref[b].astype(jnp.bfloat16),
                 v_ref[b].astype(jnp.bfloat16))

    partners = lambda c: (yp, xp) if c % 2 == 0 else (xp, yp)

    accs = [None] * len(CHUNKS)
    r1 = [None] * len(CHUNKS)
    r2 = [None] * len(CHUNKS)
    sums = [None] * len(CHUNKS)

    def launch_r2(c):
        r1[c].wait_recv()
        sums[c] = accs[c] + comm_ref[4 * c + 1].astype(jnp.float32)
        comm_ref[4 * c + 2] = sums[c].astype(jnp.bfloat16)
        r2[c] = _exchange(comm_ref, send_sems, recv_sems,
                          4 * c + 2, 4 * c + 3, 2 * c + 1, partners(c)[1])

    for c, (b, r) in enumerate(CHUNKS):
        rows = pl.ds(b * Sq + r * R, R)
        q_c = jnp.dot(x_ref[rows, :].astype(jnp.bfloat16), wq,
                      preferred_element_type=jnp.float32)
        q_c = (q_c * SCALE).astype(jnp.bfloat16)
        accs[c] = _attn_chunk(q_c, kv[b][0], kv[b][1], wo,
                              bias[r * R:(r + 1) * R, :])
        comm_ref[4 * c + 0] = accs[c].astype(jnp.bfloat16)
        if c == 0:
            pl.semaphore_wait(barrier_sem, 2)
        r1[c] = _exchange(comm_ref, send_sems, recv_sems,
                          4 * c + 0, 4 * c + 1, 2 * c, partners(c)[0])
        if c >= 2:
            launch_r2(c - 2)
    launch_r2(len(CHUNKS) - 2)
    launch_r2(len(CHUNKS) - 1)

    for c, (b, r) in enumerate(CHUNKS):
        r2[c].wait_recv()
        out_ref[b, pl.ds(r * R, R), :] = (
            sums[c] + comm_ref[4 * c + 3].astype(jnp.float32))

    for rd in r1 + r2:
        rd.wait_send()


def kernel(x, Wq, K_ext, V_ext, Wo):
    my = lax.axis_index("i")
    K2 = lax.dynamic_slice_in_dim(K_ext, my * H, H, axis=2).reshape(
        B, Skv, H * Dh)
    V2 = lax.dynamic_slice_in_dim(V_ext, my * H, H, axis=2).reshape(
        B, Skv, H * Dh)
    x2 = x.reshape(B * Sq, D_MODEL)

    n_ex = 2 * len(CHUNKS)
    return pl.pallas_call(
        _body,
        out_shape=jax.ShapeDtypeStruct((B, Sq, D_MODEL), jnp.float32),
        in_specs=[pl.BlockSpec(memory_space=pltpu.VMEM)] * 5,
        out_specs=pl.BlockSpec(memory_space=pltpu.VMEM),
        scratch_shapes=[
            pltpu.VMEM((4 * len(CHUNKS), R, D_MODEL), jnp.bfloat16),
            pltpu.SemaphoreType.DMA((n_ex,)),
            pltpu.SemaphoreType.DMA((n_ex,)),
        ],
        compiler_params=pltpu.CompilerParams(collective_id=0),
    )(x2, Wq, K2, V2, Wo)
